# baseline (speedup 1.0000x reference)
import functools

import jax
import jax.numpy as jnp
from jax import lax
from jax.experimental import pallas as pl
from jax.experimental.pallas import tpu as pltpu
from jax.experimental.pallas import tpu_sc as plsc

N_NODES = 10000
D_FEAT = 128
D_COUNT = 16
D_IN = 2 * D_FEAT + 2 * D_COUNT
D_X = D_FEAT + D_COUNT
WIDE = D_X
OUT = 128

NC, NS = 2, 16
NW = NC * NS
CHUNK = 20
NCH = 500
NPAD = 10240
ZROWS = 8


def _mm_body(na_ref, sc_ref, w_ref, o1_ref, o2_ref):
    x = jnp.concatenate([na_ref[...], sc_ref[...]], axis=1)
    w = w_ref[...]
    wsrc = jnp.concatenate([w[0:D_FEAT], w[2 * D_FEAT:2 * D_FEAT + D_COUNT]],
                           axis=0)
    wdst = jnp.concatenate([w[D_FEAT:2 * D_FEAT], w[2 * D_FEAT + D_COUNT:]],
                           axis=0)
    y = jnp.dot(x, jnp.concatenate([wsrc, wdst], axis=1),
                preferred_element_type=jnp.float32)
    r = x.shape[0]
    col = lax.broadcasted_iota(jnp.int32, (r, D_COUNT), 1)
    ones_col = jnp.where(col == 0, 1.0, 0.0).astype(jnp.float32)
    o1_ref[...] = jnp.concatenate([y[:, :OUT], ones_col], axis=1)
    o2_ref[...] = y[:, OUT:]


def _matmul_pre(node_attr, sub_counting, weight):
    n = node_attr.shape[0]
    r = 1000
    return pl.pallas_call(
        _mm_body,
        grid=(n // r,),
        in_specs=[
            pl.BlockSpec((r, D_FEAT), lambda i: (i, 0)),
            pl.BlockSpec((r, D_COUNT), lambda i: (i, 0)),
            pl.BlockSpec((D_IN, OUT), lambda i: (0, 0)),
        ],
        out_specs=[
            pl.BlockSpec((r, WIDE), lambda i: (i, 0)),
            pl.BlockSpec((r, OUT), lambda i: (i, 0)),
        ],
        out_shape=[
            jax.ShapeDtypeStruct((n, WIDE), jnp.float32),
            jax.ShapeDtypeStruct((n, OUT), jnp.float32),
        ],
    )(node_attr, sub_counting, weight)


def _sc_body(ysrc_hbm, src_hbm, dst_hbm, zero_hbm, out_hbm,
             sidx, didx, rows0, rows1, rows2, rows3, acc,
             sg0, sg1, sg2, sg3, ss0, ss1, ss2, ss3):
    c = lax.axis_index("c")
    s = lax.axis_index("s")
    wid = s * NC + c
    bufs = (rows0, rows1, rows2, rows3)
    gs = (sg0, sg1, sg2, sg3)
    ss = (ss0, ss1, ss2, ss3)

    def g_issue(t, buf, sem):
        pltpu.async_copy(ysrc_hbm.at[sidx.at[t]], buf, sem)

    def g_wait(buf, sem):
        pltpu.make_async_copy(ysrc_hbm.at[sidx.at[0]], buf, sem).wait()

    def s_issue(t, buf, sem):
        pltpu.async_copy(buf, acc.at[didx.at[t]], sem, add=True)

    def s_wait(buf, sem):
        pltpu.make_async_copy(buf, acc.at[didx.at[0]], sem).wait()

    ld_s = pltpu.async_copy(src_hbm.at[wid], sidx, sg0)
    ld_d = pltpu.async_copy(dst_hbm.at[wid], didx, sg1)
    rps = NPAD // NS
    pltpu.sync_copy(zero_hbm.at[pl.ds(s * rps, rps)],
                    acc.at[pl.ds(s * rps, rps)])
    ld_s.wait()
    ld_d.wait()
    plsc.subcore_barrier()

    g_issue(0, rows0, sg0)
    g_issue(1, rows1, sg1)
    g_wait(rows0, sg0); s_issue(0, rows0, ss0); g_issue(2, rows2, sg2)
    g_wait(rows1, sg1); s_issue(1, rows1, ss1); g_issue(3, rows3, sg3)
    g_wait(rows2, sg2); s_issue(2, rows2, ss2)
    s_wait(rows0, ss0); g_issue(4, rows0, sg0)
    g_wait(rows3, sg3); s_issue(3, rows3, ss3)
    s_wait(rows1, ss1); g_issue(5, rows1, sg1)
    g_wait(rows0, sg0); s_issue(4, rows0, ss0)
    s_wait(rows2, ss2); g_issue(6, rows2, sg2)
    g_wait(rows1, sg1); s_issue(5, rows1, ss1)
    s_wait(rows3, ss3); g_issue(7, rows3, sg3)

    def body4(j, carry):
        t0 = 6 + 4 * j
        for k in range(4):
            u = (2 + k) % 4
            v = (u + 2) % 4
            g_wait(bufs[u], gs[u])
            s_issue(t0 + k, bufs[u], ss[u])
            s_wait(bufs[v], ss[v])
            g_issue(t0 + k + 2, bufs[v], gs[v])
        return carry
    lax.fori_loop(0, (NCH - 8) // 4, body4, 0)

    g_wait(rows2, sg2); s_issue(NCH - 2, rows2, ss2); s_wait(rows0, ss0)
    g_wait(rows3, sg3); s_issue(NCH - 1, rows3, ss3); s_wait(rows1, ss1)
    s_wait(rows2, ss2)
    s_wait(rows3, ss3)
    plsc.subcore_barrier()

    pltpu.sync_copy(acc.at[pl.ds(s * rps, rps)],
                    out_hbm.at[c, pl.ds(s * rps, rps)])


@functools.partial(jax.jit, static_argnames=())
def _sc_scatter(ysrc_wide, src, dst, zero):
    mesh = plsc.VectorSubcoreMesh(core_axis_name="c", subcore_axis_name="s")
    f = pl.kernel(
        _sc_body,
        out_type=jax.ShapeDtypeStruct((NC, NPAD, WIDE), jnp.float32),
        mesh=mesh,
        scratch_types=[
            pltpu.VMEM((NCH, CHUNK), jnp.int32),
            pltpu.VMEM((NCH, CHUNK), jnp.int32),
            pltpu.VMEM((CHUNK, WIDE), jnp.float32),
            pltpu.VMEM((CHUNK, WIDE), jnp.float32),
            pltpu.VMEM((CHUNK, WIDE), jnp.float32),
            pltpu.VMEM((CHUNK, WIDE), jnp.float32),
            pltpu.VMEM_SHARED((NPAD, WIDE), jnp.float32),
        ] + [pltpu.SemaphoreType.DMA] * 8,
        compiler_params=pltpu.CompilerParams(use_tc_tiling_on_sc=False),
    )
    return f(ysrc_wide, src, dst, zero)


def _comb_body(p_ref, y_ref, b_ref, o_ref):
    p = p_ref[...]
    s = p[0] + p[1]
    agg = s[:, :OUT]
    deg = s[:, OUT:OUT + 1]
    r = (agg + deg * y_ref[...]) * lax.rsqrt(jnp.maximum(deg, 1.0))
    o_ref[...] = jnp.maximum(r + b_ref[...], 0.0)


def _combine(parts, ydst, bias2d):
    n = ydst.shape[0]
    r = 1000
    return pl.pallas_call(
        _comb_body,
        grid=(n // r,),
        in_specs=[
            pl.BlockSpec((NC, r, WIDE), lambda i: (0, i, 0)),
            pl.BlockSpec((r, OUT), lambda i: (i, 0)),
            pl.BlockSpec((1, OUT), lambda i: (0, 0)),
        ],
        out_specs=pl.BlockSpec((r, OUT), lambda i: (i, 0)),
        out_shape=jax.ShapeDtypeStruct((n, OUT), jnp.float32),
    )(parts, ydst, bias2d)


def kernel(node_attr, sub_counting, edge_index, weight, bias):
    ei = edge_index.astype(jnp.int32)
    src = ei[0].reshape(NW, NCH, CHUNK)
    dst = ei[1].reshape(NW, NCH, CHUNK)
    ysrc_wide, ydst = _matmul_pre(node_attr, sub_counting, weight)
    zero = jnp.zeros((NPAD, WIDE), jnp.float32)
    parts = _sc_scatter(ysrc_wide, src, dst, zero)
    return _combine(parts, ydst, bias.reshape(1, OUT))


# speedup vs baseline: 1.4345x; 1.4345x over previous
import functools

import jax
import jax.numpy as jnp
from jax import lax
from jax.experimental import pallas as pl
from jax.experimental.pallas import tpu as pltpu
from jax.experimental.pallas import tpu_sc as plsc

N_NODES = 10000
D_FEAT = 128
D_COUNT = 16
D_IN = 2 * D_FEAT + 2 * D_COUNT
D_X = D_FEAT + D_COUNT
WIDE = D_X
OUT = 128

NC, NS = 2, 16
NW = NC * NS
CHUNK = 40
NCH = 250
NPAD = 10240
ZROWS = 8


def _mm_body(na_ref, sc_ref, w_ref, o1_ref, o2_ref):
    x = jnp.concatenate([na_ref[...], sc_ref[...]], axis=1)
    w = w_ref[...]
    wsrc = jnp.concatenate([w[0:D_FEAT], w[2 * D_FEAT:2 * D_FEAT + D_COUNT]],
                           axis=0)
    wdst = jnp.concatenate([w[D_FEAT:2 * D_FEAT], w[2 * D_FEAT + D_COUNT:]],
                           axis=0)
    y = jnp.dot(x, jnp.concatenate([wsrc, wdst], axis=1),
                preferred_element_type=jnp.float32)
    r = x.shape[0]
    col = lax.broadcasted_iota(jnp.int32, (r, D_COUNT), 1)
    ones_col = jnp.where(col == 0, 1.0, 0.0).astype(jnp.float32)
    o1_ref[...] = jnp.concatenate([y[:, :OUT], ones_col], axis=1)
    o2_ref[...] = y[:, OUT:]


def _matmul_pre(node_attr, sub_counting, weight):
    n = node_attr.shape[0]
    r = 2000
    return pl.pallas_call(
        _mm_body,
        grid=(n // r,),
        in_specs=[
            pl.BlockSpec((r, D_FEAT), lambda i: (i, 0)),
            pl.BlockSpec((r, D_COUNT), lambda i: (i, 0)),
            pl.BlockSpec((D_IN, OUT), lambda i: (0, 0)),
        ],
        out_specs=[
            pl.BlockSpec((r, WIDE), lambda i: (i, 0)),
            pl.BlockSpec((r, OUT), lambda i: (i, 0)),
        ],
        out_shape=[
            jax.ShapeDtypeStruct((n, WIDE), jnp.float32),
            jax.ShapeDtypeStruct((n, OUT), jnp.float32),
        ],
    )(node_attr, sub_counting, weight)


def _sc_body(ysrc_hbm, eidx_hbm, out_hbm,
             sidx, didx, rows0, rows1, zbuf, acc, sem_a, sem_b):
    c = lax.axis_index("c")
    s = lax.axis_index("s")
    wid = s * NC + c

    ld_s = pltpu.async_copy(eidx_hbm.at[0, wid], sidx, sem_a)
    ld_d = pltpu.async_copy(eidx_hbm.at[1, wid], didx, sem_b)

    def zfill(rr, carry):
        for j in range(WIDE // 16):
            zbuf[rr, pl.ds(j * 16, 16)] = jnp.zeros((16,), jnp.float32)
        return carry
    lax.fori_loop(0, ZROWS, zfill, 0)
    rps = NPAD // NS
    def zcopy(k, carry):
        pltpu.sync_copy(zbuf, acc.at[pl.ds(s * rps + k * ZROWS, ZROWS)])
        return carry
    lax.fori_loop(0, rps // ZROWS, zcopy, 0)
    ld_s.wait()
    ld_d.wait()
    plsc.subcore_barrier()

    pltpu.async_copy(ysrc_hbm.at[sidx.at[0]], rows0, sem_a)
    pltpu.async_copy(ysrc_hbm.at[sidx.at[1]], rows1, sem_b)

    def body2(j, carry):
        i = 2 * j
        pltpu.make_async_copy(ysrc_hbm.at[sidx.at[i]], rows0, sem_a).wait()
        pltpu.sync_copy(rows0, acc.at[didx.at[i]], add=True)
        pltpu.async_copy(ysrc_hbm.at[sidx.at[i + 2]], rows0, sem_a)
        pltpu.make_async_copy(ysrc_hbm.at[sidx.at[i + 1]], rows1, sem_b).wait()
        pltpu.sync_copy(rows1, acc.at[didx.at[i + 1]], add=True)
        pltpu.async_copy(ysrc_hbm.at[sidx.at[i + 3]], rows1, sem_b)
        return carry
    lax.fori_loop(0, NCH // 2 - 1, body2, 0)

    i = NCH - 2
    pltpu.make_async_copy(ysrc_hbm.at[sidx.at[i]], rows0, sem_a).wait()
    pltpu.sync_copy(rows0, acc.at[didx.at[i]], add=True)
    pltpu.make_async_copy(ysrc_hbm.at[sidx.at[i + 1]], rows1, sem_b).wait()
    pltpu.sync_copy(rows1, acc.at[didx.at[i + 1]], add=True)
    plsc.subcore_barrier()

    pltpu.sync_copy(acc.at[pl.ds(s * rps, rps)],
                    out_hbm.at[c, pl.ds(s * rps, rps)])


@functools.partial(jax.jit, static_argnames=())
def _sc_scatter(ysrc_wide, eidx):
    mesh = plsc.VectorSubcoreMesh(core_axis_name="c", subcore_axis_name="s")
    f = pl.kernel(
        _sc_body,
        out_type=jax.ShapeDtypeStruct((NC, NPAD, WIDE), jnp.float32),
        mesh=mesh,
        scratch_types=[
            pltpu.VMEM((NCH, CHUNK), jnp.int32),
            pltpu.VMEM((NCH, CHUNK), jnp.int32),
            pltpu.VMEM((CHUNK, WIDE), jnp.float32),
            pltpu.VMEM((CHUNK, WIDE), jnp.float32),
            pltpu.VMEM((ZROWS, WIDE), jnp.float32),
            pltpu.VMEM_SHARED((NPAD, WIDE), jnp.float32),
            pltpu.SemaphoreType.DMA,
            pltpu.SemaphoreType.DMA,
        ],
        compiler_params=pltpu.CompilerParams(use_tc_tiling_on_sc=False),
    )
    return f(ysrc_wide, eidx)


def _comb_body(p_ref, y_ref, b_ref, o_ref):
    p = p_ref[...]
    s = p[0] + p[1]
    agg = s[:, :OUT]
    deg = s[:, OUT:OUT + 1]
    r = (agg + deg * y_ref[...]) * lax.rsqrt(jnp.maximum(deg, 1.0))
    o_ref[...] = jnp.maximum(r + b_ref[...], 0.0)


def _combine(parts, ydst, bias2d):
    n = ydst.shape[0]
    r = 2000
    return pl.pallas_call(
        _comb_body,
        grid=(n // r,),
        in_specs=[
            pl.BlockSpec((NC, r, WIDE), lambda i: (0, i, 0)),
            pl.BlockSpec((r, OUT), lambda i: (i, 0)),
            pl.BlockSpec((1, OUT), lambda i: (0, 0)),
        ],
        out_specs=pl.BlockSpec((r, OUT), lambda i: (i, 0)),
        out_shape=jax.ShapeDtypeStruct((n, OUT), jnp.float32),
    )(parts, ydst, bias2d)


def kernel(node_attr, sub_counting, edge_index, weight, bias):
    eidx = edge_index.astype(jnp.int32).reshape(2, NW, NCH, CHUNK)
    ysrc_wide, ydst = _matmul_pre(node_attr, sub_counting, weight)
    parts = _sc_scatter(ysrc_wide, eidx)
    return _combine(parts, ydst, bias.reshape(1, OUT))
